# baseline (speedup 1.0000x reference)
import functools

import jax
import jax.numpy as jnp
from jax import lax
from jax.experimental import pallas as pl
from jax.experimental.pallas import tpu as pltpu
from jax.experimental.pallas import tpu_sc as plsc

NC = 2
NS = 16
NW = NC * NS
LANES = 16

EMBED_DIM = 64
HIST = 50
PAD_HIST = 56
PAD_DIM = 128
NBUF = 4


def _body(idx_hbm, table_hbm, out_hbm, idx_v, rows_v, zrow_v, gsem, wsem,
          zsem):
    batches = idx_hbm.shape[0] // NW
    wid = lax.axis_index("s") * NC + lax.axis_index("c")

    pltpu.sync_copy(idx_hbm.at[pl.ds(wid * batches, batches)], idx_v)

    zeros16 = jnp.zeros((LANES,), jnp.float32)

    def gather(i, b):
        return pltpu.make_async_copy(
            table_hbm.at[idx_v.at[i]], rows_v.at[b], gsem.at[b])

    def write(i, b):
        return pltpu.make_async_copy(
            rows_v.at[b],
            out_hbm.at[wid * batches + i, pl.ds(0, HIST), pl.ds(0, EMBED_DIM)],
            wsem.at[b])

    for b in range(NBUF):
        gather(b, b).start()

    koffs = [0, 16, HIST - LANES]

    def scan_row(r, acc):
        for ko in koffs:
            iv = idx_v[r, pl.ds(ko, LANES)]
            acc = acc | jnp.where(iv == 0, jnp.int32(1), jnp.int32(0))
        return acc

    acc = lax.fori_loop(0, batches, scan_row, jnp.zeros((LANES,), jnp.int32))
    wflag = acc[0]
    for e in range(1, LANES):
        wflag = wflag | acc[e]
    worker_has_zero = wflag > 0

    nblk = batches // NBUF

    def block(g, carry):
        for b in range(NBUF):
            i = g * NBUF + b
            gather(i, b).wait()
            write(i, b).start()

            @pl.when(g < nblk - 1)
            def _next(i=i, b=b):
                write(i, b).wait()
                gather(i + NBUF, b).start()
        return carry

    lax.fori_loop(0, nblk, block, jnp.int32(0))

    for b in range(NBUF):
        write((nblk - 1) * NBUF + b, b).wait()

    @pl.when(worker_has_zero)
    def _fixup():
        for q in range(0, EMBED_DIM, LANES):
            zrow_v[0, pl.ds(q, LANES)] = zeros16

        def row(i, c):
            for ko in koffs:
                iv = idx_v[i, pl.ds(ko, LANES)]
                zm = jnp.where(iv == 0, jnp.int32(1), jnp.int32(0))
                for e in range(LANES):
                    @pl.when(zm[e] > 0)
                    def _zero_row(ko=ko, e=e):
                        pltpu.async_copy(
                            zrow_v,
                            out_hbm.at[wid * batches + i,
                                       pl.ds(ko + e, 1),
                                       pl.ds(0, EMBED_DIM)],
                            zsem).wait()
            return c

        lax.fori_loop(0, batches, row, jnp.int32(0))


def kernel(words, table):
    B, H = words.shape
    V, D = table.shape
    assert D == EMBED_DIM and H == HIST
    batches = B // NW
    assert B % NW == 0 and batches % NBUF == 0

    idx = words.astype(jnp.int32)

    mesh = plsc.VectorSubcoreMesh(core_axis_name="c", subcore_axis_name="s")

    run = functools.partial(
        pl.kernel,
        out_type=jax.ShapeDtypeStruct((B, PAD_HIST, PAD_DIM), jnp.float32),
        mesh=mesh,
        compiler_params=pltpu.CompilerParams(use_tc_tiling_on_sc=False),
        scratch_types=[
            pltpu.VMEM((batches, HIST), jnp.int32),
            pltpu.VMEM((NBUF, HIST, EMBED_DIM), jnp.float32),
            pltpu.VMEM((1, EMBED_DIM), jnp.float32),
            pltpu.SemaphoreType.DMA((NBUF,)),
            pltpu.SemaphoreType.DMA((NBUF,)),
            pltpu.SemaphoreType.DMA,
        ],
    )(_body)

    outp = run(idx, table)
    return outp[:, :HIST, :EMBED_DIM]


# speedup vs baseline: 1.0213x; 1.0213x over previous
import functools

import jax
import jax.numpy as jnp
from jax import lax
from jax.experimental import pallas as pl
from jax.experimental.pallas import tpu as pltpu
from jax.experimental.pallas import tpu_sc as plsc

NC = 2
NS = 16
NW = NC * NS
LANES = 16

EMBED_DIM = 64
HIST = 50
PAD_HIST = 56
PAD_DIM = 128
NBUF = 8


def _body(idx_hbm, table_hbm, out_hbm, idx_v, rows_v, zrow_v, gsem, wsem,
          zsem):
    batches = idx_hbm.shape[0] // NW
    wid = lax.axis_index("s") * NC + lax.axis_index("c")

    pltpu.sync_copy(idx_hbm.at[pl.ds(wid * batches, batches)], idx_v)

    zeros16 = jnp.zeros((LANES,), jnp.float32)

    def gather(i, b):
        return pltpu.make_async_copy(
            table_hbm.at[idx_v.at[i]], rows_v.at[b], gsem.at[b])

    def write(i, b):
        return pltpu.make_async_copy(
            rows_v.at[b],
            out_hbm.at[wid * batches + i, pl.ds(0, HIST), pl.ds(0, EMBED_DIM)],
            wsem.at[b])

    for b in range(NBUF):
        gather(b, b).start()

    koffs = [0, 16, HIST - LANES]

    def scan_row(r, acc):
        for ko in koffs:
            iv = idx_v[r, pl.ds(ko, LANES)]
            acc = acc | jnp.where(iv == 0, jnp.int32(1), jnp.int32(0))
        return acc

    acc = lax.fori_loop(0, batches, scan_row, jnp.zeros((LANES,), jnp.int32))
    wflag = acc[0]
    for e in range(1, LANES):
        wflag = wflag | acc[e]
    worker_has_zero = wflag > 0

    nblk = batches // NBUF

    def block(g, carry):
        for b in range(NBUF):
            i = g * NBUF + b
            gather(i, b).wait()
            write(i, b).start()

            @pl.when(g < nblk - 1)
            def _next(i=i, b=b):
                write(i, b).wait()
                gather(i + NBUF, b).start()
        return carry

    lax.fori_loop(0, nblk, block, jnp.int32(0))

    for b in range(NBUF):
        write((nblk - 1) * NBUF + b, b).wait()

    @pl.when(worker_has_zero)
    def _fixup():
        for q in range(0, EMBED_DIM, LANES):
            zrow_v[0, pl.ds(q, LANES)] = zeros16

        def row(i, c):
            for ko in koffs:
                iv = idx_v[i, pl.ds(ko, LANES)]
                zm = jnp.where(iv == 0, jnp.int32(1), jnp.int32(0))
                for e in range(LANES):
                    @pl.when(zm[e] > 0)
                    def _zero_row(ko=ko, e=e):
                        pltpu.async_copy(
                            zrow_v,
                            out_hbm.at[wid * batches + i,
                                       pl.ds(ko + e, 1),
                                       pl.ds(0, EMBED_DIM)],
                            zsem).wait()
            return c

        lax.fori_loop(0, batches, row, jnp.int32(0))


def kernel(words, table):
    B, H = words.shape
    V, D = table.shape
    assert D == EMBED_DIM and H == HIST
    batches = B // NW
    assert B % NW == 0 and batches % NBUF == 0

    idx = words.astype(jnp.int32)

    mesh = plsc.VectorSubcoreMesh(core_axis_name="c", subcore_axis_name="s")

    run = functools.partial(
        pl.kernel,
        out_type=jax.ShapeDtypeStruct((B, PAD_HIST, PAD_DIM), jnp.float32),
        mesh=mesh,
        compiler_params=pltpu.CompilerParams(use_tc_tiling_on_sc=False),
        scratch_types=[
            pltpu.VMEM((batches, HIST), jnp.int32),
            pltpu.VMEM((NBUF, HIST, EMBED_DIM), jnp.float32),
            pltpu.VMEM((1, EMBED_DIM), jnp.float32),
            pltpu.SemaphoreType.DMA((NBUF,)),
            pltpu.SemaphoreType.DMA((NBUF,)),
            pltpu.SemaphoreType.DMA,
        ],
    )(_body)

    outp = run(idx, table)
    return outp[:, :HIST, :EMBED_DIM]


# speedup vs baseline: 1.4504x; 1.4202x over previous
import functools

import jax
import jax.numpy as jnp
from jax import lax
from jax.experimental import pallas as pl
from jax.experimental.pallas import tpu as pltpu
from jax.experimental.pallas import tpu_sc as plsc

NC = 2
NS = 16
NW = NC * NS
LANES = 16

EMBED_DIM = 64
HIST = 50
PAD_HIST = 56
PAD_DIM = 128
NBUF = 8


def _body(idx_hbm, table_hbm, out_hbm, idx_v, rows_v, gsem, wsem):
    batches = idx_hbm.shape[0] // NW
    wid = lax.axis_index("s") * NC + lax.axis_index("c")

    pltpu.sync_copy(idx_hbm.at[pl.ds(wid * batches, batches)], idx_v)

    zeros16 = jnp.zeros((LANES,), jnp.float32)

    def gather(i, b):
        return pltpu.make_async_copy(
            table_hbm.at[idx_v.at[i]], rows_v.at[b], gsem.at[b])

    def write(i, b):
        return pltpu.make_async_copy(
            rows_v.at[b],
            out_hbm.at[wid * batches + i, pl.ds(0, HIST), pl.ds(0, EMBED_DIM)],
            wsem.at[b])

    for b in range(NBUF):
        gather(b, b).start()

    koffs = [0, 16, HIST - LANES]

    def scan_row(r, acc):
        for ko in koffs:
            iv = idx_v[r, pl.ds(ko, LANES)]
            acc = acc | jnp.where(iv == 0, jnp.int32(1), jnp.int32(0))
        return acc

    acc = lax.fori_loop(0, batches, scan_row, jnp.zeros((LANES,), jnp.int32))
    wflag = acc[0]
    for e in range(1, LANES):
        wflag = wflag | acc[e]
    worker_has_zero = wflag > 0

    def fixup(i, b):
        @pl.when(worker_has_zero)
        def _check():
            cacc = jnp.zeros((LANES,), jnp.int32)
            for ko in koffs:
                iv = idx_v[i, pl.ds(ko, LANES)]
                cacc = cacc | jnp.where(iv == 0, jnp.int32(1), jnp.int32(0))
            flag = cacc[0]
            for e in range(1, LANES):
                flag = flag | cacc[e]

            @pl.when(flag > 0)
            def _do():
                for ko in koffs:
                    iv = idx_v[i, pl.ds(ko, LANES)]
                    zm = jnp.where(iv == 0, jnp.int32(1), jnp.int32(0))
                    for e in range(LANES):
                        @pl.when(zm[e] > 0)
                        def _zero_row(ko=ko, e=e):
                            for col in range(0, EMBED_DIM, LANES):
                                rows_v[b, ko + e, pl.ds(col, LANES)] = zeros16

    nblk = batches // NBUF

    def block(g, carry):
        for b in range(NBUF):
            i = g * NBUF + b
            gather(i, b).wait()
            fixup(i, b)
            write(i, b).start()

            @pl.when(g < nblk - 1)
            def _next(i=i, b=b):
                write(i, b).wait()
                gather(i + NBUF, b).start()
        return carry

    lax.fori_loop(0, nblk, block, jnp.int32(0))

    for b in range(NBUF):
        write((nblk - 1) * NBUF + b, b).wait()


def kernel(words, table):
    B, H = words.shape
    V, D = table.shape
    assert D == EMBED_DIM and H == HIST
    batches = B // NW
    assert B % NW == 0 and batches % NBUF == 0

    idx = words.astype(jnp.int32)

    mesh = plsc.VectorSubcoreMesh(core_axis_name="c", subcore_axis_name="s")

    run = functools.partial(
        pl.kernel,
        out_type=jax.ShapeDtypeStruct((B, PAD_HIST, PAD_DIM), jnp.float32),
        mesh=mesh,
        compiler_params=pltpu.CompilerParams(use_tc_tiling_on_sc=False),
        scratch_types=[
            pltpu.VMEM((batches, HIST), jnp.int32),
            pltpu.VMEM((NBUF, HIST, EMBED_DIM), jnp.float32),
            pltpu.SemaphoreType.DMA((NBUF,)),
            pltpu.SemaphoreType.DMA((NBUF,)),
        ],
    )(_body)

    outp = run(idx, table)
    return outp[:, :HIST, :EMBED_DIM]


# speedup vs baseline: 1.4544x; 1.0028x over previous
import functools

import jax
import jax.numpy as jnp
from jax import lax
from jax.experimental import pallas as pl
from jax.experimental.pallas import tpu as pltpu
from jax.experimental.pallas import tpu_sc as plsc

NC = 2
NS = 16
NW = NC * NS
LANES = 16

EMBED_DIM = 64
HIST = 50
PAD_HIST = 56
PAD_DIM = 128
NBUF = 8


def _body(idx_hbm, table_hbm, out_hbm, idx_v, rows_v, gsem):
    batches = idx_hbm.shape[0] // NW
    wid = lax.axis_index("s") * NC + lax.axis_index("c")

    pltpu.sync_copy(idx_hbm.at[pl.ds(wid * batches, batches)], idx_v)

    zeros16 = jnp.zeros((LANES,), jnp.float32)

    def gather(i, b):
        return pltpu.make_async_copy(
            table_hbm.at[idx_v.at[i]], rows_v.at[b], gsem.at[b])

    def write(i, b):
        return pltpu.make_async_copy(
            rows_v.at[b],
            out_hbm.at[wid * batches + i, pl.ds(0, HIST), pl.ds(0, EMBED_DIM)],
            gsem.at[b])

    for b in range(NBUF):
        gather(b, b).start()

    koffs = [0, 16, HIST - LANES]

    def scan_row(r, acc):
        for ko in koffs:
            iv = idx_v[r, pl.ds(ko, LANES)]
            acc = acc | jnp.where(iv == 0, jnp.int32(1), jnp.int32(0))
        return acc

    acc = lax.fori_loop(0, batches, scan_row, jnp.zeros((LANES,), jnp.int32))
    wflag = acc[0]
    for e in range(1, LANES):
        wflag = wflag | acc[e]
    worker_has_zero = wflag > 0

    def fixup(i, b):
        @pl.when(worker_has_zero)
        def _check():
            cacc = jnp.zeros((LANES,), jnp.int32)
            for ko in koffs:
                iv = idx_v[i, pl.ds(ko, LANES)]
                cacc = cacc | jnp.where(iv == 0, jnp.int32(1), jnp.int32(0))
            flag = cacc[0]
            for e in range(1, LANES):
                flag = flag | cacc[e]

            @pl.when(flag > 0)
            def _do():
                for ko in koffs:
                    iv = idx_v[i, pl.ds(ko, LANES)]
                    zm = jnp.where(iv == 0, jnp.int32(1), jnp.int32(0))
                    for e in range(LANES):
                        @pl.when(zm[e] > 0)
                        def _zero_row(ko=ko, e=e):
                            for col in range(0, EMBED_DIM, LANES):
                                rows_v[b, ko + e, pl.ds(col, LANES)] = zeros16

    nblk = batches // NBUF

    def block(g, carry):
        for b in range(NBUF):
            i = g * NBUF + b
            gather(i, b).wait()
            fixup(i, b)
            write(i, b).start()

            @pl.when(g < nblk - 1)
            def _next(i=i, b=b):
                write(i, b).wait()
                gather(i + NBUF, b).start()
        return carry

    lax.fori_loop(0, nblk, block, jnp.int32(0))

    for b in range(NBUF):
        write((nblk - 1) * NBUF + b, b).wait()


def kernel(words, table):
    B, H = words.shape
    V, D = table.shape
    assert D == EMBED_DIM and H == HIST
    batches = B // NW
    assert B % NW == 0 and batches % NBUF == 0

    idx = words.astype(jnp.int32)

    mesh = plsc.VectorSubcoreMesh(core_axis_name="c", subcore_axis_name="s")

    run = functools.partial(
        pl.kernel,
        out_type=jax.ShapeDtypeStruct((B, PAD_HIST, PAD_DIM), jnp.float32),
        mesh=mesh,
        compiler_params=pltpu.CompilerParams(use_tc_tiling_on_sc=False),
        scratch_types=[
            pltpu.VMEM((batches, HIST), jnp.int32),
            pltpu.VMEM((NBUF, HIST, EMBED_DIM), jnp.float32),
            pltpu.SemaphoreType.DMA((NBUF,)),
        ],
    )(_body)

    outp = run(idx, table)
    return outp[:, :HIST, :EMBED_DIM]
